# Initial kernel scaffold; baseline (speedup 1.0000x reference)
import jax
import jax.numpy as jnp
from jax.experimental import pallas as pl


def kernel(xyz, features, params):
    raise NotImplementedError("write your pallas kernel here")


# speedup vs baseline: 1.0028x; 1.0028x over previous
import functools
import jax
import jax.numpy as jnp
import numpy as np
from jax.experimental import pallas as pl
from jax.experimental.pallas import tpu as pltpu

NPOINT = 2048
RADIUS_LIST = [0.1, 0.2, 0.4]
NSAMPLE_LIST = [16, 32, 64]
EPS = 1e-5
TR = 2048
LANES = 128


def _pass_kernel(x_ref, wt_ref, vec_ref, y_ref, sums_ref, *, norm, kmax, do_sums):
    step = pl.program_id(0)
    h = x_ref[...]
    if norm:
        mean = vec_ref[1, :][None, :]
        rstd = vec_ref[2, :][None, :]
        gamma = vec_ref[3, :][None, :]
        beta = vec_ref[4, :][None, :]
        alpha = vec_ref[5, :][None, :]
        h = (h - mean) * rstd * gamma + beta
        h = jnp.where(h > 0, h, alpha * h)
    if wt_ref is not None:
        y = jnp.dot(h, wt_ref[...], preferred_element_type=jnp.float32)
        y = y + vec_ref[0, :][None, :]
    else:
        y = h
    if kmax is not None:
        y = y.reshape(y.shape[0] // kmax, kmax, LANES).max(axis=1)
    y_ref[...] = y
    if do_sums:
        @pl.when(step == 0)
        def _init():
            sums_ref[...] = jnp.zeros_like(sums_ref)
        s1 = jnp.sum(y, axis=0)
        s2 = jnp.sum(y * y, axis=0)
        sums_ref[...] += jnp.stack([s1, s2], axis=0)


def _mlp_pass(x, wt, vecs, *, norm, kmax, do_sums):
    R, Cin = x.shape
    grid = R // TR
    out_rows = TR // kmax if kmax is not None else TR
    out_shapes = [jax.ShapeDtypeStruct((grid * out_rows, LANES), jnp.float32),
                  jax.ShapeDtypeStruct((2, LANES), jnp.float32)]
    in_specs = [pl.BlockSpec((TR, Cin), lambda i: (i, 0))]
    operands = [x]
    if wt is not None:
        in_specs.append(pl.BlockSpec((Cin, LANES), lambda i: (0, 0)))
        operands.append(wt)
    in_specs.append(pl.BlockSpec((8, LANES), lambda i: (0, 0)))
    operands.append(vecs)
    out_specs = [pl.BlockSpec((out_rows, LANES), lambda i: (i, 0)),
                 pl.BlockSpec((2, LANES), lambda i: (0, 0))]

    def body(*refs):
        if wt is None:
            x_ref, vec_ref, y_ref, sums_ref = refs
            wt_ref = None
        else:
            x_ref, wt_ref, vec_ref, y_ref, sums_ref = refs
        _pass_kernel(x_ref, wt_ref, vec_ref, y_ref, sums_ref,
                     norm=norm, kmax=kmax, do_sums=do_sums)

    y, sums = pl.pallas_call(
        body,
        grid=(grid,),
        in_specs=in_specs,
        out_specs=out_specs,
        out_shape=out_shapes,
    )(*operands)
    return y, (sums if do_sums else None)


def _pack_vecs(b, mean, rstd, gamma, beta, alpha):
    rows = []
    for v in (b, mean, rstd, gamma, beta):
        rows.append(jnp.pad(v, (0, LANES - v.shape[0])) if v is not None
                    else jnp.zeros((LANES,), jnp.float32))
    rows.append(jnp.full((LANES,), alpha, jnp.float32) if alpha is not None
                else jnp.zeros((LANES,), jnp.float32))
    rows.append(jnp.zeros((LANES,), jnp.float32))
    rows.append(jnp.zeros((LANES,), jnp.float32))
    return jnp.stack(rows, axis=0)


def _pad_wt(w, cin_pad):
    cout, cin = w.shape
    return jnp.pad(w.T, ((0, cin_pad - cin), (0, LANES - cout)))


def _stats(sums, n_rows):
    s1, s2 = sums[0], sums[1]
    mean = s1 / n_rows
    var = s2 / n_rows - mean * mean
    rstd = 1.0 / jnp.sqrt(var + EPS)
    return mean, rstd


def _run_layers(x, layers, n_rows, kmax_last):
    vecs0 = _pack_vecs(jnp.pad(layers[0]['b'],
                               (0, LANES - layers[0]['b'].shape[0])),
                       None, None, None, None, None)
    y, sums = _mlp_pass(x, _pad_wt(layers[0]['w'], x.shape[1]), vecs0,
                        norm=False, kmax=None, do_sums=True)
    for li in range(1, len(layers) + 1):
        prev = layers[li - 1]
        c = prev['w'].shape[0]
        mean, rstd = _stats(sums, n_rows)
        gamma = jnp.pad(prev['gamma'], (0, LANES - c))
        beta = jnp.pad(prev['beta'], (0, LANES - c))
        if li < len(layers):
            layer = layers[li]
            vecs = _pack_vecs(jnp.pad(layer['b'], (0, LANES - layer['w'].shape[0])),
                              mean, rstd, gamma, beta, prev['a'][0])
            y, sums = _mlp_pass(y, _pad_wt(layer['w'], y.shape[1]), vecs,
                                norm=True, kmax=None, do_sums=True)
        else:
            vecs = _pack_vecs(None, mean, rstd, gamma, beta, prev['a'][0])
            y, _ = _mlp_pass(y, None, vecs,
                             norm=True, kmax=kmax_last, do_sums=False)
    return y


def _square_distance(src, dst):
    dist = -2.0 * jnp.matmul(src, jnp.swapaxes(dst, 1, 2))
    dist = dist + jnp.sum(src ** 2, axis=-1)[:, :, None]
    dist = dist + jnp.sum(dst ** 2, axis=-1)[:, None, :]
    return dist


def kernel(xyz, features, params):
    B, N, _ = xyz.shape
    S = NPOINT
    C_in = features.shape[-1]

    keys = jax.random.split(jax.random.key(42), B)
    fps_idx = jax.vmap(lambda k: jax.random.permutation(k, N))(keys)[:, :S]
    new_xyz = jax.vmap(lambda p, i: p[i])(xyz, fps_idx)

    sqrdists = _square_distance(new_xyz, xyz)

    branch_outs = []
    for i, radius in enumerate(RADIUS_LIST):
        K = NSAMPLE_LIST[i]
        val = jnp.where(sqrdists > radius ** 2,
                        jnp.int32(N),
                        jnp.broadcast_to(jnp.arange(N, dtype=jnp.int32),
                                         sqrdists.shape))
        neg_topk, _ = jax.lax.top_k(-val, K)
        group_idx = -neg_topk
        group_first = group_idx[:, :, 0:1]
        group_idx = jnp.where(group_idx == N, group_first, group_idx)

        grouped_xyz = jax.vmap(lambda p, ii: p[ii])(xyz, group_idx)
        grouped_xyz = grouped_xyz - new_xyz[:, :, None, :]
        grouped_feat = jax.vmap(lambda p, ii: p[ii])(features, group_idx)
        grouped = jnp.concatenate([grouped_feat, grouped_xyz], axis=-1)
        R = B * S * K
        x = grouped.reshape(R, C_in + 3)
        x = jnp.pad(x, ((0, 0), (0, 32 - x.shape[1])))

        y = _run_layers(x, params['branches'][i], R, kmax_last=K)
        c_out = params['branches'][i][-1]['w'].shape[0]
        branch_outs.append(y[:, :c_out])

    cat = jnp.concatenate(branch_outs, axis=-1)
    cat = jnp.pad(cat, ((0, 0), (0, 256 - cat.shape[1])))
    yf = _run_layers(cat, params['final'], B * S, kmax_last=None)
    c_f = params['final'][-1]['w'].shape[0]
    new_features = yf[:, :c_f].reshape(B, S, c_f).transpose(0, 2, 1)
    return new_xyz, new_features


# speedup vs baseline: 2.2032x; 2.1970x over previous
import functools
import jax
import jax.numpy as jnp
import numpy as np
from jax.experimental import pallas as pl
from jax.experimental.pallas import tpu as pltpu

NPOINT = 2048
RADIUS_LIST = [0.1, 0.2, 0.4]
NSAMPLE_LIST = [16, 32, 64]
EPS = 1e-5
TR = 2048
LANES = 128


def _pass_kernel(x_ref, wt_ref, vec_ref, y_ref, sums_ref, *, norm, kmax, do_sums):
    step = pl.program_id(0)
    h = x_ref[...]
    if norm:
        mean = vec_ref[1, :][None, :]
        rstd = vec_ref[2, :][None, :]
        gamma = vec_ref[3, :][None, :]
        beta = vec_ref[4, :][None, :]
        alpha = vec_ref[5, :][None, :]
        h = (h - mean) * rstd * gamma + beta
        h = jnp.where(h > 0, h, alpha * h)
    if wt_ref is not None:
        y = jnp.dot(h, wt_ref[...], preferred_element_type=jnp.float32)
        y = y + vec_ref[0, :][None, :]
    else:
        y = h
    if kmax is not None:
        y = y.reshape(y.shape[0] // kmax, kmax, LANES).max(axis=1)
    y_ref[...] = y
    if do_sums:
        @pl.when(step == 0)
        def _init():
            sums_ref[...] = jnp.zeros_like(sums_ref)
        s1 = jnp.sum(y, axis=0)
        s2 = jnp.sum(y * y, axis=0)
        sums_ref[...] += jnp.stack([s1, s2], axis=0)


def _mlp_pass(x, wt, vecs, *, norm, kmax, do_sums):
    R, Cin = x.shape
    grid = R // TR
    out_rows = TR // kmax if kmax is not None else TR
    out_shapes = [jax.ShapeDtypeStruct((grid * out_rows, LANES), jnp.float32),
                  jax.ShapeDtypeStruct((2, LANES), jnp.float32)]
    in_specs = [pl.BlockSpec((TR, Cin), lambda i: (i, 0))]
    operands = [x]
    if wt is not None:
        in_specs.append(pl.BlockSpec((Cin, LANES), lambda i: (0, 0)))
        operands.append(wt)
    in_specs.append(pl.BlockSpec((8, LANES), lambda i: (0, 0)))
    operands.append(vecs)
    out_specs = [pl.BlockSpec((out_rows, LANES), lambda i: (i, 0)),
                 pl.BlockSpec((2, LANES), lambda i: (0, 0))]

    def body(*refs):
        if wt is None:
            x_ref, vec_ref, y_ref, sums_ref = refs
            wt_ref = None
        else:
            x_ref, wt_ref, vec_ref, y_ref, sums_ref = refs
        _pass_kernel(x_ref, wt_ref, vec_ref, y_ref, sums_ref,
                     norm=norm, kmax=kmax, do_sums=do_sums)

    y, sums = pl.pallas_call(
        body,
        grid=(grid,),
        in_specs=in_specs,
        out_specs=out_specs,
        out_shape=out_shapes,
    )(*operands)
    return y, (sums if do_sums else None)


def _pack_vecs(b, mean, rstd, gamma, beta, alpha):
    rows = []
    for v in (b, mean, rstd, gamma, beta):
        rows.append(jnp.pad(v, (0, LANES - v.shape[0])) if v is not None
                    else jnp.zeros((LANES,), jnp.float32))
    rows.append(jnp.full((LANES,), alpha, jnp.float32) if alpha is not None
                else jnp.zeros((LANES,), jnp.float32))
    rows.append(jnp.zeros((LANES,), jnp.float32))
    rows.append(jnp.zeros((LANES,), jnp.float32))
    return jnp.stack(rows, axis=0)


def _pad_wt(w, cin_pad):
    cout, cin = w.shape
    return jnp.pad(w.T, ((0, cin_pad - cin), (0, LANES - cout)))


def _stats(sums, n_rows):
    s1, s2 = sums[0], sums[1]
    mean = s1 / n_rows
    var = s2 / n_rows - mean * mean
    rstd = 1.0 / jnp.sqrt(var + EPS)
    return mean, rstd


def _run_layers(x, layers, n_rows, kmax_last):
    vecs0 = _pack_vecs(jnp.pad(layers[0]['b'],
                               (0, LANES - layers[0]['b'].shape[0])),
                       None, None, None, None, None)
    y, sums = _mlp_pass(x, _pad_wt(layers[0]['w'], x.shape[1]), vecs0,
                        norm=False, kmax=None, do_sums=True)
    for li in range(1, len(layers) + 1):
        prev = layers[li - 1]
        c = prev['w'].shape[0]
        mean, rstd = _stats(sums, n_rows)
        gamma = jnp.pad(prev['gamma'], (0, LANES - c))
        beta = jnp.pad(prev['beta'], (0, LANES - c))
        if li < len(layers):
            layer = layers[li]
            vecs = _pack_vecs(jnp.pad(layer['b'], (0, LANES - layer['w'].shape[0])),
                              mean, rstd, gamma, beta, prev['a'][0])
            y, sums = _mlp_pass(y, _pad_wt(layer['w'], y.shape[1]), vecs,
                                norm=True, kmax=None, do_sums=True)
        else:
            vecs = _pack_vecs(None, mean, rstd, gamma, beta, prev['a'][0])
            y, _ = _mlp_pass(y, None, vecs,
                             norm=True, kmax=kmax_last, do_sums=False)
    return y


TS = 256


def _ballquery_kernel(nx_ref, xt_ref, o0_ref, o1_ref, o2_ref):
    s = nx_ref[0]
    xt = xt_ref[0]
    n = xt.shape[1]
    dist = -2.0 * jnp.dot(s, xt, preferred_element_type=jnp.float32)
    dist = dist + jnp.sum(s * s, axis=1, keepdims=True)
    dist = dist + jnp.sum(xt * xt, axis=0, keepdims=True)
    iota = jax.lax.broadcasted_iota(jnp.int32, (1, n), 1).astype(jnp.float32)
    big = jnp.float32(3.0 * n)
    fn = jnp.float32(n)
    for radius, k_samp, o_ref in ((RADIUS_LIST[0], NSAMPLE_LIST[0], o0_ref),
                                  (RADIUS_LIST[1], NSAMPLE_LIST[1], o1_ref),
                                  (RADIUS_LIST[2], NSAMPLE_LIST[2], o2_ref)):
        val = jnp.where(dist > radius ** 2, fn, iota)
        cols = []
        first = None
        for _ in range(k_samp):
            m = jnp.min(val, axis=1, keepdims=True)
            if first is None:
                first = m
            cols.append(jnp.where(m >= fn, first, m))
            val = jnp.where(val == m, big, val)
        o_ref[0] = jnp.concatenate(cols, axis=1).astype(jnp.int32)


def _ball_query(new_xyz, xyz):
    B, S, _ = new_xyz.shape
    N = xyz.shape[1]
    xt = xyz.transpose(0, 2, 1)
    return pl.pallas_call(
        _ballquery_kernel,
        grid=(B, S // TS),
        in_specs=[pl.BlockSpec((1, TS, 3), lambda b, t: (b, t, 0)),
                  pl.BlockSpec((1, 3, N), lambda b, t: (b, 0, 0))],
        out_specs=[pl.BlockSpec((1, TS, k), lambda b, t: (b, t, 0))
                   for k in NSAMPLE_LIST],
        out_shape=[jax.ShapeDtypeStruct((B, S, k), jnp.int32)
                   for k in NSAMPLE_LIST],
    )(new_xyz, xt)


def kernel(xyz, features, params):
    B, N, _ = xyz.shape
    S = NPOINT
    C_in = features.shape[-1]

    keys = jax.random.split(jax.random.key(42), B)
    fps_idx = jax.vmap(lambda k: jax.random.permutation(k, N))(keys)[:, :S]
    new_xyz = jax.vmap(lambda p, i: p[i])(xyz, fps_idx)

    group_idx_all = _ball_query(new_xyz, xyz)

    branch_outs = []
    for i, radius in enumerate(RADIUS_LIST):
        K = NSAMPLE_LIST[i]
        group_idx = group_idx_all[i]

        grouped_xyz = jax.vmap(lambda p, ii: p[ii])(xyz, group_idx)
        grouped_xyz = grouped_xyz - new_xyz[:, :, None, :]
        grouped_feat = jax.vmap(lambda p, ii: p[ii])(features, group_idx)
        grouped = jnp.concatenate([grouped_feat, grouped_xyz], axis=-1)
        R = B * S * K
        x = grouped.reshape(R, C_in + 3)
        x = jnp.pad(x, ((0, 0), (0, 32 - x.shape[1])))

        y = _run_layers(x, params['branches'][i], R, kmax_last=K)
        c_out = params['branches'][i][-1]['w'].shape[0]
        branch_outs.append(y[:, :c_out])

    cat = jnp.concatenate(branch_outs, axis=-1)
    cat = jnp.pad(cat, ((0, 0), (0, 256 - cat.shape[1])))
    yf = _run_layers(cat, params['final'], B * S, kmax_last=None)
    c_f = params['final'][-1]['w'].shape[0]
    new_features = yf[:, :c_f].reshape(B, S, c_f).transpose(0, 2, 1)
    return new_xyz, new_features


# speedup vs baseline: 8.3329x; 3.7821x over previous
import functools
import jax
import jax.numpy as jnp
import numpy as np
from jax.experimental import pallas as pl
from jax.experimental.pallas import tpu as pltpu

NPOINT = 2048
RADIUS_LIST = [0.1, 0.2, 0.4]
NSAMPLE_LIST = [16, 32, 64]
EPS = 1e-5
TR = 2048
LANES = 128


def _pass_kernel(x_ref, wt_ref, vec_ref, y_ref, sums_ref, *, norm, kmax, do_sums):
    step = pl.program_id(0)
    h = x_ref[...]
    if norm:
        mean = vec_ref[1, :][None, :]
        rstd = vec_ref[2, :][None, :]
        gamma = vec_ref[3, :][None, :]
        beta = vec_ref[4, :][None, :]
        alpha = vec_ref[5, :][None, :]
        h = (h - mean) * rstd * gamma + beta
        h = jnp.where(h > 0, h, alpha * h)
    if wt_ref is not None:
        y = jnp.dot(h, wt_ref[...], preferred_element_type=jnp.float32)
        y = y + vec_ref[0, :][None, :]
    else:
        y = h
    if kmax is not None:
        y = y.reshape(y.shape[0] // kmax, kmax, LANES).max(axis=1)
    y_ref[...] = y
    if do_sums:
        @pl.when(step == 0)
        def _init():
            sums_ref[...] = jnp.zeros_like(sums_ref)
        s1 = jnp.sum(y, axis=0)
        s2 = jnp.sum(y * y, axis=0)
        sums_ref[...] += jnp.stack([s1, s2], axis=0)


def _mlp_pass(x, wt, vecs, *, norm, kmax, do_sums):
    R, Cin = x.shape
    grid = R // TR
    out_rows = TR // kmax if kmax is not None else TR
    out_shapes = [jax.ShapeDtypeStruct((grid * out_rows, LANES), jnp.float32),
                  jax.ShapeDtypeStruct((2, LANES), jnp.float32)]
    in_specs = [pl.BlockSpec((TR, Cin), lambda i: (i, 0))]
    operands = [x]
    if wt is not None:
        in_specs.append(pl.BlockSpec((Cin, LANES), lambda i: (0, 0)))
        operands.append(wt)
    in_specs.append(pl.BlockSpec((8, LANES), lambda i: (0, 0)))
    operands.append(vecs)
    out_specs = [pl.BlockSpec((out_rows, LANES), lambda i: (i, 0)),
                 pl.BlockSpec((2, LANES), lambda i: (0, 0))]

    def body(*refs):
        if wt is None:
            x_ref, vec_ref, y_ref, sums_ref = refs
            wt_ref = None
        else:
            x_ref, wt_ref, vec_ref, y_ref, sums_ref = refs
        _pass_kernel(x_ref, wt_ref, vec_ref, y_ref, sums_ref,
                     norm=norm, kmax=kmax, do_sums=do_sums)

    y, sums = pl.pallas_call(
        body,
        grid=(grid,),
        in_specs=in_specs,
        out_specs=out_specs,
        out_shape=out_shapes,
    )(*operands)
    return y, (sums if do_sums else None)


def _pack_vecs(b, mean, rstd, gamma, beta, alpha):
    rows = []
    for v in (b, mean, rstd, gamma, beta):
        rows.append(jnp.pad(v, (0, LANES - v.shape[0])) if v is not None
                    else jnp.zeros((LANES,), jnp.float32))
    rows.append(jnp.full((LANES,), alpha, jnp.float32) if alpha is not None
                else jnp.zeros((LANES,), jnp.float32))
    rows.append(jnp.zeros((LANES,), jnp.float32))
    rows.append(jnp.zeros((LANES,), jnp.float32))
    return jnp.stack(rows, axis=0)


def _pad_wt(w, cin_pad):
    cout, cin = w.shape
    return jnp.pad(w.T, ((0, cin_pad - cin), (0, LANES - cout)))


def _stats(sums, n_rows):
    s1, s2 = sums[0], sums[1]
    mean = s1 / n_rows
    var = s2 / n_rows - mean * mean
    rstd = 1.0 / jnp.sqrt(var + EPS)
    return mean, rstd


def _run_layers(x, layers, n_rows, kmax_last):
    vecs0 = _pack_vecs(jnp.pad(layers[0]['b'],
                               (0, LANES - layers[0]['b'].shape[0])),
                       None, None, None, None, None)
    y, sums = _mlp_pass(x, _pad_wt(layers[0]['w'], x.shape[1]), vecs0,
                        norm=False, kmax=None, do_sums=True)
    for li in range(1, len(layers) + 1):
        prev = layers[li - 1]
        c = prev['w'].shape[0]
        mean, rstd = _stats(sums, n_rows)
        gamma = jnp.pad(prev['gamma'], (0, LANES - c))
        beta = jnp.pad(prev['beta'], (0, LANES - c))
        if li < len(layers):
            layer = layers[li]
            vecs = _pack_vecs(jnp.pad(layer['b'], (0, LANES - layer['w'].shape[0])),
                              mean, rstd, gamma, beta, prev['a'][0])
            y, sums = _mlp_pass(y, _pad_wt(layer['w'], y.shape[1]), vecs,
                                norm=True, kmax=None, do_sums=True)
        else:
            vecs = _pack_vecs(None, mean, rstd, gamma, beta, prev['a'][0])
            y, _ = _mlp_pass(y, None, vecs,
                             norm=True, kmax=kmax_last, do_sums=False)
    return y


TS = 256


def _ballquery_kernel(d_ref, o0_ref, o1_ref, o2_ref):
    dist = d_ref[0]
    n = dist.shape[1]
    iota = jax.lax.broadcasted_iota(jnp.int32, (1, n), 1).astype(jnp.float32)
    big = jnp.float32(3.0 * n)
    fn = jnp.float32(n)
    for radius, k_samp, o_ref in ((RADIUS_LIST[0], NSAMPLE_LIST[0], o0_ref),
                                  (RADIUS_LIST[1], NSAMPLE_LIST[1], o1_ref),
                                  (RADIUS_LIST[2], NSAMPLE_LIST[2], o2_ref)):
        val = jnp.where(dist > radius ** 2, fn, iota)
        cols = []
        first = None
        for _ in range(k_samp):
            m = jnp.min(val, axis=1, keepdims=True)
            if first is None:
                first = m
            cols.append(jnp.where(m >= fn, first, m))
            val = jnp.where(val == m, big, val)
        o_ref[0] = jnp.concatenate(cols, axis=1).astype(jnp.int32)


def _ball_query(new_xyz, xyz):
    B, S, _ = new_xyz.shape
    N = xyz.shape[1]
    sqrdists = -2.0 * jnp.matmul(new_xyz, jnp.swapaxes(xyz, 1, 2))
    sqrdists = sqrdists + jnp.sum(new_xyz ** 2, axis=-1)[:, :, None]
    sqrdists = sqrdists + jnp.sum(xyz ** 2, axis=-1)[:, None, :]
    return pl.pallas_call(
        _ballquery_kernel,
        grid=(B, S // TS),
        in_specs=[pl.BlockSpec((1, TS, N), lambda b, t: (b, t, 0))],
        out_specs=[pl.BlockSpec((1, TS, k), lambda b, t: (b, t, 0))
                   for k in NSAMPLE_LIST],
        out_shape=[jax.ShapeDtypeStruct((B, S, k), jnp.int32)
                   for k in NSAMPLE_LIST],
    )(sqrdists)


def kernel(xyz, features, params):
    B, N, _ = xyz.shape
    S = NPOINT
    C_in = features.shape[-1]

    keys = jax.random.split(jax.random.key(42), B)
    fps_idx = jax.vmap(lambda k: jax.random.permutation(k, N))(keys)[:, :S]
    new_xyz = jax.vmap(lambda p, i: p[i])(xyz, fps_idx)

    group_idx_all = _ball_query(new_xyz, xyz)

    table = jnp.concatenate(
        [features, xyz, jnp.zeros((B, N, 32 - C_in - 3), jnp.float32)],
        axis=-1).reshape(B * N, 32)
    base = (jnp.arange(B, dtype=jnp.int32) * N)[:, None, None]

    branch_outs = []
    for i, radius in enumerate(RADIUS_LIST):
        K = NSAMPLE_LIST[i]
        R = B * S * K
        gidx = (group_idx_all[i] + base).reshape(R)
        rows = jnp.take(table, gidx, axis=0)
        center = jnp.broadcast_to(new_xyz[:, :, None, :], (B, S, K, 3))
        cpad = jnp.pad(center.reshape(R, 3), ((0, 0), (C_in, 32 - C_in - 3)))
        x = rows - cpad

        y = _run_layers(x, params['branches'][i], R, kmax_last=K)
        c_out = params['branches'][i][-1]['w'].shape[0]
        branch_outs.append(y[:, :c_out])

    cat = jnp.concatenate(branch_outs, axis=-1)
    cat = jnp.pad(cat, ((0, 0), (0, 256 - cat.shape[1])))
    yf = _run_layers(cat, params['final'], B * S, kmax_last=None)
    c_f = params['final'][-1]['w'].shape[0]
    new_features = yf[:, :c_f].reshape(B, S, c_f).transpose(0, 2, 1)
    return new_xyz, new_features
